# baseline (speedup 1.0000x reference)
import functools

import jax
import jax.numpy as jnp
from jax import lax
from jax.experimental import pallas as pl
from jax.experimental.pallas import tpu as pltpu
from jax.experimental.pallas import tpu_sc as plsc

TM = 128


def _router_body(x_ref, gw_ref, pos_ref, meta_ref):
    x = x_ref[...]
    gw = gw_ref[...]
    T, _ = x.shape
    E = gw.shape[0]
    Gp = meta_ref.shape[0]
    logits = lax.dot_general(x, gw, (((1,), (1,)), ((), ())),
                             preferred_element_type=jnp.float32)
    amax = jnp.max(logits, axis=1, keepdims=True)
    col = lax.broadcasted_iota(jnp.int32, (T, E), 1)
    eid = jnp.min(jnp.where(logits >= amax, col, E), axis=1)
    onehot = (col == eid[:, None]).astype(jnp.float32)
    TB = 256
    r = lax.broadcasted_iota(jnp.int32, (TB, TB), 0)
    c = lax.broadcasted_iota(jnp.int32, (TB, TB), 1)
    tri = (r >= c).astype(jnp.float32)
    carry = jnp.zeros((1, E), jnp.float32)
    blocks = []
    for i in range(T // TB):
        oh = onehot[i * TB:(i + 1) * TB, :]
        cs = lax.dot_general(tri, oh, (((1,), (0,)), ((), ())),
                             preferred_element_type=jnp.float32)
        blocks.append(cs + carry)
        carry = carry + jnp.sum(oh, axis=0)[None, :]
    csum = jnp.concatenate(blocks, axis=0)
    rank = jnp.sum(onehot * csum, axis=1) - 1.0
    counts = carry
    er = lax.broadcasted_iota(jnp.int32, (E, E), 0)
    ec = lax.broadcasted_iota(jnp.int32, (E, E), 1)
    stri = (er < ec).astype(jnp.float32)
    off = lax.dot_general(counts, stri, (((1,), (0,)), ((), ())),
                          preferred_element_type=jnp.float32)
    base = jnp.sum(onehot * off, axis=1)
    pos_ref[...] = (base + rank).astype(jnp.int32)

    cnt_i = counts.astype(jnp.int32)
    off_i = off.astype(jnp.int32)
    csum_i = off_i + cnt_i
    t_start = off_i // TM
    t_last = (csum_i - 1) // TM
    p = jnp.where(cnt_i > 0, t_last - t_start + 1, 0)
    itri = (er <= ec).astype(jnp.float32)
    P = lax.dot_general(p.astype(jnp.float32), itri,
                        (((1,), (0,)), ((), ())),
                        preferred_element_type=jnp.float32).astype(jnp.int32)
    total = P[:, E - 1:E]
    g = lax.broadcasted_iota(jnp.int32, (Gp, 1), 0)
    gv = jnp.minimum(g, total - 1)
    eg = jnp.sum((P <= gv).astype(jnp.int32), axis=1,
                 keepdims=True)
    eoh = (lax.broadcasted_iota(jnp.int32, (Gp, E), 1) ==
           eg).astype(jnp.int32)
    Pprev_g = jnp.sum(eoh * (P - p), axis=1, keepdims=True)
    ts_g = jnp.sum(eoh * t_start, axis=1, keepdims=True)
    off_g = jnp.sum(eoh * off_i, axis=1, keepdims=True)
    cnt_g = jnp.sum(eoh * cnt_i, axis=1, keepdims=True)
    m = ts_g + (gv - Pprev_g)
    rs = jnp.maximum(off_g - m * TM, 0)
    re = jnp.minimum(off_g + cnt_g - m * TM, TM)
    valid = g < total
    rs = jnp.where(valid, rs, 0)
    re = jnp.where(valid, re, 0)
    first = (valid & (rs == 0)).astype(jnp.int32)
    meta_ref[...] = jnp.concatenate([eg, m, rs, re, first], axis=1)


def _router(x, gate_w, Gp):
    T = x.shape[0]
    return pl.pallas_call(
        _router_body,
        out_shape=[
            jax.ShapeDtypeStruct((T,), jnp.int32),
            jax.ShapeDtypeStruct((Gp, 5), jnp.int32),
        ],
    )(x, gate_w)


TS = 256


def _gmm_fused(meta, xs, ew_gate, ew_up, ew_down, x, wgu, wdn, segw):
    T, H = xs.shape
    E, DFF, _ = ew_gate.shape
    G = T // TM + E - 1
    NTS = T // TS
    SFF = wgu.shape[0] // 2

    def body(meta_ref, xs_ref, wg_ref, wu_ref, wd_ref,
             x_ref, wgu_ref, wdn_ref, segw_ref, out_ref, sh_ref):
        g = pl.program_id(0)
        rs = meta_ref[g, 2]
        re = meta_ref[g, 3]
        first = meta_ref[g, 4]
        xb = xs_ref[...]
        hg = lax.dot_general(xb, wg_ref[0], (((1,), (1,)), ((), ())),
                             preferred_element_type=jnp.float32,
                             precision=lax.Precision.DEFAULT)
        hu = lax.dot_general(xb, wu_ref[0], (((1,), (1,)), ((), ())),
                             preferred_element_type=jnp.float32,
                             precision=lax.Precision.DEFAULT)
        h = hg * jax.nn.sigmoid(hg) * hu
        o = lax.dot_general(h, wd_ref[0], (((1,), (1,)), ((), ())),
                            preferred_element_type=jnp.float32,
                            precision=lax.Precision.DEFAULT)
        rows = lax.broadcasted_iota(jnp.int32, (TM, 1), 0)
        mask = (rows >= rs) & (rows < re)

        @pl.when(first == 1)
        def _():
            out_ref[...] = jnp.where(mask, o, 0.0)

        @pl.when(first == 0)
        def _():
            out_ref[...] = jnp.where(mask, o, out_ref[...])

        @pl.when(g < NTS)
        def _():
            xt = x_ref[...]
            gu = lax.dot_general(xt, wgu_ref[...], (((1,), (1,)), ((), ())),
                                 preferred_element_type=jnp.float32,
                                 precision=lax.Precision.DEFAULT)
            a = gu[:, :SFF]
            b = gu[:, SFF:]
            sh = a * jax.nn.sigmoid(a) * b
            so = lax.dot_general(sh, wdn_ref[...], (((1,), (1,)), ((), ())),
                                 preferred_element_type=jnp.float32,
                                 precision=lax.Precision.DEFAULT)
            gate = jax.nn.sigmoid(
                lax.dot_general(xt, segw_ref[...], (((1,), (1,)), ((), ())),
                                preferred_element_type=jnp.float32))
            sh_ref[...] = gate * so

    grid_spec = pltpu.PrefetchScalarGridSpec(
        num_scalar_prefetch=1,
        grid=(G,),
        in_specs=[
            pl.BlockSpec((TM, H), lambda g, meta: (meta[g, 1], 0)),
            pl.BlockSpec((1, DFF, H), lambda g, meta: (meta[g, 0], 0, 0)),
            pl.BlockSpec((1, DFF, H), lambda g, meta: (meta[g, 0], 0, 0)),
            pl.BlockSpec((1, H, DFF), lambda g, meta: (meta[g, 0], 0, 0)),
            pl.BlockSpec((TS, H), lambda g, meta: (jnp.minimum(g, NTS - 1), 0)),
            pl.BlockSpec(wgu.shape, lambda g, meta: (0, 0)),
            pl.BlockSpec(wdn.shape, lambda g, meta: (0, 0)),
            pl.BlockSpec(segw.shape, lambda g, meta: (0, 0)),
        ],
        out_specs=[
            pl.BlockSpec((TM, H), lambda g, meta: (meta[g, 1], 0)),
            pl.BlockSpec((TS, H), lambda g, meta: (jnp.minimum(g, NTS - 1), 0)),
        ],
    )
    return pl.pallas_call(
        body,
        grid_spec=grid_spec,
        out_shape=[
            jax.ShapeDtypeStruct((T, H), jnp.float32),
            jax.ShapeDtypeStruct((T, H), jnp.float32),
        ],
    )(meta, xs, ew_gate, ew_up, ew_down, x, wgu, wdn, segw)


def _sc_permute(rows_in, pos, direction):
    T, H = rows_in.shape
    info = plsc.get_sparse_core_info()
    NC = info.num_cores
    NW = NC * info.num_subcores
    bpw = T // NW
    mesh = plsc.VectorSubcoreMesh(core_axis_name="c", subcore_axis_name="s")

    @functools.partial(
        pl.kernel, mesh=mesh,
        out_type=jax.ShapeDtypeStruct((T, H), jnp.float32),
        scratch_types=[
            pltpu.VMEM((bpw,), jnp.int32),
            pltpu.VMEM((bpw, H), jnp.float32),
            pltpu.SemaphoreType.DMA,
        ],
    )
    def k(src_hbm, pos_hbm, out_hbm, idx_v, rows_v, sem):
        wid = lax.axis_index("s") * NC + lax.axis_index("c")
        base = wid * bpw
        pltpu.sync_copy(pos_hbm.at[pl.ds(base, bpw)], idx_v)
        if direction == "scatter":
            pltpu.sync_copy(src_hbm.at[pl.ds(base, bpw)], rows_v)
            pltpu.async_copy(rows_v, out_hbm.at[idx_v], sem).wait()
        else:
            pltpu.async_copy(src_hbm.at[idx_v], rows_v, sem).wait()
            pltpu.sync_copy(rows_v, out_hbm.at[pl.ds(base, bpw)])

    return k(rows_in, pos)


def _shared(x, sh_gate_up, sh_down, seg_w, moe):
    T, H = x.shape
    TS = 256
    SFF = sh_gate_up.shape[0] // 2

    def body(x_ref, wgu_ref, wdn_ref, segw_ref, moe_ref, out_ref):
        xb = x_ref[...]
        gu = lax.dot_general(xb, wgu_ref[...], (((1,), (1,)), ((), ())),
                             preferred_element_type=jnp.float32,
                             precision=lax.Precision.DEFAULT)
        a = gu[:, :SFF]
        b = gu[:, SFF:]
        sh = a * jax.nn.sigmoid(a) * b
        so = lax.dot_general(sh, wdn_ref[...], (((1,), (1,)), ((), ())),
                             preferred_element_type=jnp.float32,
                             precision=lax.Precision.DEFAULT)
        gate = jax.nn.sigmoid(
            lax.dot_general(xb, segw_ref[...], (((1,), (1,)), ((), ())),
                            preferred_element_type=jnp.float32))
        out_ref[...] = moe_ref[...] + gate * so

    return pl.pallas_call(
        body,
        grid=(T // TS,),
        in_specs=[
            pl.BlockSpec((TS, H), lambda i: (i, 0)),
            pl.BlockSpec(sh_gate_up.shape, lambda i: (0, 0)),
            pl.BlockSpec(sh_down.shape, lambda i: (0, 0)),
            pl.BlockSpec(seg_w.shape, lambda i: (0, 0)),
            pl.BlockSpec((TS, H), lambda i: (i, 0)),
        ],
        out_specs=pl.BlockSpec((TS, H), lambda i: (i, 0)),
        out_shape=jax.ShapeDtypeStruct((T, H), jnp.float32),
    )(x, sh_gate_up, sh_down, seg_w, moe)


def kernel(hidden_states, gate_w, ew_gate, ew_up, ew_down, sh_gate_up,
           sh_down, seg_w):
    orig_shape = hidden_states.shape
    H = orig_shape[-1]
    x = hidden_states.reshape(-1, H)
    T = x.shape[0]
    E = gate_w.shape[0]

    G = T // TM + E - 1
    pos, meta = _router(x, gate_w, G)
    xs = _sc_permute(x, pos, "scatter")
    moe_sorted, sh_out = _gmm_fused(meta, xs, ew_gate, ew_up, ew_down,
                                    x, sh_gate_up, sh_down, seg_w)
    moe = _sc_permute(moe_sorted, pos, "gather")
    out = moe + sh_out
    return out.reshape(orig_shape)


# speedup vs baseline: 1.0198x; 1.0198x over previous
import functools

import jax
import jax.numpy as jnp
from jax import lax
from jax.experimental import pallas as pl
from jax.experimental.pallas import tpu as pltpu
from jax.experimental.pallas import tpu_sc as plsc

TM = 128


def _router_body(x_ref, gw_ref, pos_ref, meta_ref):
    x = x_ref[...]
    gw = gw_ref[...]
    T, _ = x.shape
    E = gw.shape[0]
    Gp = meta_ref.shape[0]
    logits = lax.dot_general(x, gw, (((1,), (1,)), ((), ())),
                             preferred_element_type=jnp.float32)
    amax = jnp.max(logits, axis=1, keepdims=True)
    col = lax.broadcasted_iota(jnp.int32, (T, E), 1)
    eid = jnp.min(jnp.where(logits >= amax, col, E), axis=1)
    onehot = (col == eid[:, None]).astype(jnp.float32)
    TB = 256
    r = lax.broadcasted_iota(jnp.int32, (TB, TB), 0)
    c = lax.broadcasted_iota(jnp.int32, (TB, TB), 1)
    tri = (r >= c).astype(jnp.float32)
    carry = jnp.zeros((1, E), jnp.float32)
    blocks = []
    for i in range(T // TB):
        oh = onehot[i * TB:(i + 1) * TB, :]
        cs = lax.dot_general(tri, oh, (((1,), (0,)), ((), ())),
                             preferred_element_type=jnp.float32)
        blocks.append(cs + carry)
        carry = carry + jnp.sum(oh, axis=0)[None, :]
    csum = jnp.concatenate(blocks, axis=0)
    rank = jnp.sum(onehot * csum, axis=1) - 1.0
    counts = carry
    er = lax.broadcasted_iota(jnp.int32, (E, E), 0)
    ec = lax.broadcasted_iota(jnp.int32, (E, E), 1)
    stri = (er < ec).astype(jnp.float32)
    off = lax.dot_general(counts, stri, (((1,), (0,)), ((), ())),
                          preferred_element_type=jnp.float32)
    base = jnp.sum(onehot * off, axis=1)
    pos_ref[...] = (base + rank).astype(jnp.int32)

    cnt_i = counts.astype(jnp.int32)
    off_i = off.astype(jnp.int32)
    csum_i = off_i + cnt_i
    t_start = off_i // TM
    t_last = (csum_i - 1) // TM
    p = jnp.where(cnt_i > 0, t_last - t_start + 1, 0)
    itri = (er <= ec).astype(jnp.float32)
    P = lax.dot_general(p.astype(jnp.float32), itri,
                        (((1,), (0,)), ((), ())),
                        preferred_element_type=jnp.float32).astype(jnp.int32)
    total = P[:, E - 1:E]
    g = lax.broadcasted_iota(jnp.int32, (Gp, 1), 0)
    gv = jnp.minimum(g, total - 1)
    eg = jnp.sum((P <= gv).astype(jnp.int32), axis=1,
                 keepdims=True)
    eoh = (lax.broadcasted_iota(jnp.int32, (Gp, E), 1) ==
           eg).astype(jnp.int32)
    Pprev_g = jnp.sum(eoh * (P - p), axis=1, keepdims=True)
    ts_g = jnp.sum(eoh * t_start, axis=1, keepdims=True)
    off_g = jnp.sum(eoh * off_i, axis=1, keepdims=True)
    cnt_g = jnp.sum(eoh * cnt_i, axis=1, keepdims=True)
    m = ts_g + (gv - Pprev_g)
    rs = jnp.maximum(off_g - m * TM, 0)
    re = jnp.minimum(off_g + cnt_g - m * TM, TM)
    valid = g < total
    rs = jnp.where(valid, rs, 0)
    re = jnp.where(valid, re, 0)
    first = (valid & (rs == 0)).astype(jnp.int32)
    meta_ref[...] = jnp.concatenate([eg, m, rs, re, first], axis=1)


def _router(x, gate_w, Gp):
    T = x.shape[0]
    return pl.pallas_call(
        _router_body,
        out_shape=[
            jax.ShapeDtypeStruct((T,), jnp.int32),
            jax.ShapeDtypeStruct((Gp, 5), jnp.int32),
        ],
    )(x, gate_w)


def _gmm(meta, xs, ew_gate, ew_up, ew_down):
    T, H = xs.shape
    E, DFF, _ = ew_gate.shape
    G = T // TM + E - 1

    def body(meta_ref, xs_ref, wg_ref, wu_ref, wd_ref, out_ref):
        g = pl.program_id(0)
        rs = meta_ref[g, 2]
        re = meta_ref[g, 3]
        first = meta_ref[g, 4]
        xb = xs_ref[...]
        hg = lax.dot_general(xb, wg_ref[0], (((1,), (1,)), ((), ())),
                             preferred_element_type=jnp.float32,
                             precision=lax.Precision.DEFAULT)
        hu = lax.dot_general(xb, wu_ref[0], (((1,), (1,)), ((), ())),
                             preferred_element_type=jnp.float32,
                             precision=lax.Precision.DEFAULT)
        h = hg * jax.nn.sigmoid(hg) * hu
        o = lax.dot_general(h, wd_ref[0], (((1,), (1,)), ((), ())),
                            preferred_element_type=jnp.float32,
                            precision=lax.Precision.DEFAULT)
        rows = lax.broadcasted_iota(jnp.int32, (TM, 1), 0)
        mask = (rows >= rs) & (rows < re)

        @pl.when(first == 1)
        def _():
            out_ref[...] = jnp.where(mask, o, 0.0)

        @pl.when(first == 0)
        def _():
            out_ref[...] = jnp.where(mask, o, out_ref[...])

    grid_spec = pltpu.PrefetchScalarGridSpec(
        num_scalar_prefetch=1,
        grid=(G,),
        in_specs=[
            pl.BlockSpec((TM, H), lambda g, meta: (meta[g, 1], 0)),
            pl.BlockSpec((1, DFF, H), lambda g, meta: (meta[g, 0], 0, 0)),
            pl.BlockSpec((1, DFF, H), lambda g, meta: (meta[g, 0], 0, 0)),
            pl.BlockSpec((1, H, DFF), lambda g, meta: (meta[g, 0], 0, 0)),
        ],
        out_specs=pl.BlockSpec((TM, H), lambda g, meta: (meta[g, 1], 0)),
    )
    return pl.pallas_call(
        body,
        grid_spec=grid_spec,
        out_shape=jax.ShapeDtypeStruct((T, H), jnp.float32),
    )(meta, xs, ew_gate, ew_up, ew_down)


def _sc_permute(rows_in, pos, direction):
    T, H = rows_in.shape
    info = plsc.get_sparse_core_info()
    NC = info.num_cores
    NW = NC * info.num_subcores
    bpw = T // NW
    mesh = plsc.VectorSubcoreMesh(core_axis_name="c", subcore_axis_name="s")

    @functools.partial(
        pl.kernel, mesh=mesh,
        out_type=jax.ShapeDtypeStruct((T, H), jnp.float32),
        scratch_types=[
            pltpu.VMEM((bpw,), jnp.int32),
            pltpu.VMEM((bpw, H), jnp.float32),
            pltpu.SemaphoreType.DMA,
        ],
    )
    def k(src_hbm, pos_hbm, out_hbm, idx_v, rows_v, sem):
        wid = lax.axis_index("s") * NC + lax.axis_index("c")
        base = wid * bpw
        pltpu.sync_copy(pos_hbm.at[pl.ds(base, bpw)], idx_v)
        if direction == "scatter":
            pltpu.sync_copy(src_hbm.at[pl.ds(base, bpw)], rows_v)
            pltpu.async_copy(rows_v, out_hbm.at[idx_v], sem).wait()
        else:
            pltpu.async_copy(src_hbm.at[idx_v], rows_v, sem).wait()
            pltpu.sync_copy(rows_v, out_hbm.at[pl.ds(base, bpw)])

    return k(rows_in, pos)


def _shared(x, sh_gate_up, sh_down, seg_w, moe):
    T, H = x.shape
    TS = 256
    SFF = sh_gate_up.shape[0] // 2

    def body(x_ref, wgu_ref, wdn_ref, segw_ref, moe_ref, out_ref):
        xb = x_ref[...]
        gu = lax.dot_general(xb, wgu_ref[...], (((1,), (1,)), ((), ())),
                             preferred_element_type=jnp.float32,
                             precision=lax.Precision.DEFAULT)
        a = gu[:, :SFF]
        b = gu[:, SFF:]
        sh = a * jax.nn.sigmoid(a) * b
        so = lax.dot_general(sh, wdn_ref[...], (((1,), (1,)), ((), ())),
                             preferred_element_type=jnp.float32,
                             precision=lax.Precision.DEFAULT)
        gate = jax.nn.sigmoid(
            lax.dot_general(xb, segw_ref[...], (((1,), (1,)), ((), ())),
                            preferred_element_type=jnp.float32))
        out_ref[...] = moe_ref[...] + gate * so

    return pl.pallas_call(
        body,
        grid=(T // TS,),
        in_specs=[
            pl.BlockSpec((TS, H), lambda i: (i, 0)),
            pl.BlockSpec(sh_gate_up.shape, lambda i: (0, 0)),
            pl.BlockSpec(sh_down.shape, lambda i: (0, 0)),
            pl.BlockSpec(seg_w.shape, lambda i: (0, 0)),
            pl.BlockSpec((TS, H), lambda i: (i, 0)),
        ],
        out_specs=pl.BlockSpec((TS, H), lambda i: (i, 0)),
        out_shape=jax.ShapeDtypeStruct((T, H), jnp.float32),
    )(x, sh_gate_up, sh_down, seg_w, moe)


def kernel(hidden_states, gate_w, ew_gate, ew_up, ew_down, sh_gate_up,
           sh_down, seg_w):
    orig_shape = hidden_states.shape
    H = orig_shape[-1]
    x = hidden_states.reshape(-1, H)
    T = x.shape[0]
    E = gate_w.shape[0]

    G = T // TM + E - 1
    pos, meta = _router(x, gate_w, G)
    xs = _sc_permute(x, pos, "scatter")
    moe_sorted = _gmm(meta, xs, ew_gate, ew_up, ew_down)
    moe = _sc_permute(moe_sorted, pos, "gather")
    out = _shared(x, sh_gate_up, sh_down, seg_w, moe)
    return out.reshape(orig_shape)
